# Initial kernel scaffold; baseline (speedup 1.0000x reference)
import jax
import jax.numpy as jnp
from jax.experimental import pallas as pl


def kernel(x, edge_index, batch, feature_index, threshold, W1, b1, W2, b2):
    raise NotImplementedError("write your pallas kernel here")


# speedup vs baseline: 3.9945x; 3.9945x over previous
import functools

import jax
import jax.numpy as jnp
from jax import lax
from jax.experimental import pallas as pl
from jax.experimental.pallas import tpu as pltpu
from jax.experimental.pallas import tpu_sc as plsc

N_NODES = 100000
D = 128
B_SEG = 2048
HIDDEN = 64

NC, NS = 2, 16
NW = NC * NS
CHUNK = 128
N_FULL = N_NODES // CHUNK
TAIL = N_NODES - N_FULL * CHUNK
ITERS = (N_FULL + NW - 1) // NW
ROWS_PER_TILE = B_SEG // NS

_mesh = plsc.VectorSubcoreMesh(
    core_axis_name="c", subcore_axis_name="s", num_cores=NC, num_subcores=NS
)


@functools.partial(
    pl.kernel,
    out_type=jax.ShapeDtypeStruct((NC, B_SEG, D), jnp.float32),
    mesh=_mesh,
    scratch_types=[
        pltpu.VMEM((CHUNK, D), jnp.float32),
        pltpu.VMEM((CHUNK,), jnp.int32),
        pltpu.VMEM((TAIL, D), jnp.float32),
        pltpu.VMEM((TAIL,), jnp.int32),
        pltpu.VMEM_SHARED((B_SEG, D), jnp.float32),
    ],
)
def _segsum_sc(x_hbm, ids_hbm, zeros_hbm, out_hbm, xbuf, idx, xtail, idxtail, acc):
    c = lax.axis_index("c")
    s = lax.axis_index("s")
    w = c * NS + s

    pltpu.sync_copy(zeros_hbm, acc.at[pl.ds(s * ROWS_PER_TILE, ROWS_PER_TILE), :])
    plsc.subcore_barrier()

    def body(i, carry):
        chunk = w + i * NW

        @pl.when(chunk < N_FULL)
        def _():
            base = chunk * CHUNK
            pltpu.sync_copy(x_hbm.at[pl.ds(base, CHUNK), :], xbuf)
            pltpu.sync_copy(ids_hbm.at[pl.ds(base, CHUNK)], idx)
            pltpu.sync_copy(xbuf, acc.at[idx], add=True)

        return carry

    lax.fori_loop(0, ITERS, body, 0)

    @pl.when(w == 0)
    def _():
        base = N_FULL * CHUNK
        pltpu.sync_copy(x_hbm.at[pl.ds(base, TAIL), :], xtail)
        pltpu.sync_copy(ids_hbm.at[pl.ds(base, TAIL)], idxtail)
        pltpu.sync_copy(xtail, acc.at[idxtail], add=True)

    plsc.subcore_barrier()
    sl = pl.ds(s * ROWS_PER_TILE, ROWS_PER_TILE)
    pltpu.sync_copy(acc.at[sl, :], out_hbm.at[c, sl, :])


def _mlp_tc(pool_ref, f_ref, t_ref, w1x_ref, wf_ref, wt_ref, b1_ref, w2_ref,
            b2_ref, out_ref):
    xp = pool_ref[0] + pool_ref[1]
    h = jnp.dot(xp, w1x_ref[...], preferred_element_type=jnp.float32)
    h = h + f_ref[...] * wf_ref[...]
    h = h + t_ref[...] * wt_ref[...]
    h = h + b1_ref[...]
    h = jnp.where(h >= 0, h, 0.01 * h)
    out_ref[...] = (
        jnp.dot(h, w2_ref[...], preferred_element_type=jnp.float32) + b2_ref[...]
    )


def kernel(x, edge_index, batch, feature_index, threshold, W1, b1, W2, b2):
    del edge_index
    ids = batch.astype(jnp.int32)
    zeros = jnp.zeros((ROWS_PER_TILE, D), jnp.float32)
    partials = _segsum_sc(x, ids, zeros)

    w1x = W1[:D]
    wf = W1[D:D + 1]
    wt = W1[D + 1:D + 2]
    w2p = jnp.pad(W2, ((0, 0), (0, D - W2.shape[1])))
    b2p = jnp.pad(b2, (0, D - b2.shape[0]))[None, :]

    out = pl.pallas_call(
        _mlp_tc,
        out_shape=jax.ShapeDtypeStruct((B_SEG, D), jnp.float32),
    )(partials, feature_index[:, None], threshold[:, None], w1x, wf, wt,
      b1[None, :], w2p, b2p)
    return out[:, :2]


# speedup vs baseline: 5.2685x; 1.3190x over previous
import functools

import jax
import jax.numpy as jnp
from jax import lax
from jax.experimental import pallas as pl
from jax.experimental.pallas import tpu as pltpu
from jax.experimental.pallas import tpu_sc as plsc

N_NODES = 100000
D = 128
B_SEG = 2048
HIDDEN = 64

NC, NS = 2, 16
NW = NC * NS
SCAT = 128
UNIT = 384
IDR = UNIT // SCAT
N_UNITS = N_NODES // UNIT
TAIL_A = 99840
TAIL_B = 99968
TAIL_B_LEN = N_NODES - TAIL_B
MAXK = (N_UNITS + NW - 1) // NW
ROWS_PER_TILE = B_SEG // NS

_mesh = plsc.VectorSubcoreMesh(
    core_axis_name="c", subcore_axis_name="s", num_cores=NC, num_subcores=NS
)


@functools.partial(
    pl.kernel,
    out_type=jax.ShapeDtypeStruct((NC, B_SEG, D), jnp.float32),
    mesh=_mesh,
    scratch_types=[
        pltpu.VMEM((UNIT, D), jnp.float32),
        pltpu.VMEM((UNIT, D), jnp.float32),
        pltpu.VMEM((IDR, SCAT), jnp.int32),
        pltpu.VMEM((IDR, SCAT), jnp.int32),
        pltpu.VMEM((TAIL_B_LEN,), jnp.int32),
        pltpu.SemaphoreType.DMA,
        pltpu.SemaphoreType.DMA,
        pltpu.VMEM_SHARED((B_SEG, D), jnp.float32),
    ],
)
def _segsum_sc(x_hbm, ids_hbm, zeros_hbm, out_hbm,
               xb0, xb1, id0, id1, idxt, sem0, sem1, acc):
    c = lax.axis_index("c")
    s = lax.axis_index("s")
    w = c * NS + s

    pltpu.sync_copy(zeros_hbm, acc.at[pl.ds(s * ROWS_PER_TILE, ROWS_PER_TILE), :])
    plsc.subcore_barrier()

    def start(u, xb, idb, sem):
        pltpu.async_copy(x_hbm.at[pl.ds(u * UNIT, UNIT), :], xb, sem)
        for j in range(IDR):
            pltpu.async_copy(ids_hbm.at[pl.ds(u * UNIT + j * SCAT, SCAT)],
                             idb.at[j], sem)

    def finish(u, xb, idb, sem):
        pltpu.make_async_copy(x_hbm.at[pl.ds(u * UNIT, UNIT), :], xb, sem).wait()
        for j in range(IDR):
            pltpu.make_async_copy(ids_hbm.at[pl.ds(u * UNIT + j * SCAT, SCAT)],
                                  idb.at[j], sem).wait()
        for j in range(IDR):
            pltpu.sync_copy(xb.at[pl.ds(j * SCAT, SCAT), :],
                            acc.at[idb.at[j]], add=True)

    start(w, xb0, id0, sem0)

    def body2(k2, carry):
        u_a = w + (2 * k2) * NW
        u_b = u_a + NW
        u_c = u_b + NW

        @pl.when(u_b < N_UNITS)
        def _():
            start(u_b, xb1, id1, sem1)

        @pl.when(u_a < N_UNITS)
        def _():
            finish(u_a, xb0, id0, sem0)

        @pl.when(u_c < N_UNITS)
        def _():
            start(u_c, xb0, id0, sem0)

        @pl.when(u_b < N_UNITS)
        def _():
            finish(u_b, xb1, id1, sem1)

        return carry

    lax.fori_loop(0, (MAXK + 1) // 2, body2, 0)

    @pl.when(w == 0)
    def _():
        pltpu.sync_copy(x_hbm.at[pl.ds(TAIL_A, SCAT), :], xb0.at[pl.ds(0, SCAT), :])
        pltpu.sync_copy(ids_hbm.at[pl.ds(TAIL_A, SCAT)], id0.at[0])
        pltpu.sync_copy(xb0.at[pl.ds(0, SCAT), :], acc.at[id0.at[0]], add=True)

    @pl.when(w == 1)
    def _():
        pltpu.sync_copy(x_hbm.at[pl.ds(TAIL_B, TAIL_B_LEN), :],
                        xb1.at[pl.ds(0, TAIL_B_LEN), :])
        pltpu.sync_copy(ids_hbm.at[pl.ds(TAIL_B, TAIL_B_LEN)], idxt)
        pltpu.sync_copy(xb1.at[pl.ds(0, TAIL_B_LEN), :], acc.at[idxt], add=True)

    plsc.subcore_barrier()
    sl = pl.ds(s * ROWS_PER_TILE, ROWS_PER_TILE)
    pltpu.sync_copy(acc.at[sl, :], out_hbm.at[c, sl, :])


def _mlp_tc(pool_ref, f_ref, t_ref, w1x_ref, wf_ref, wt_ref, b1_ref, w2_ref,
            b2_ref, out_ref):
    xp = pool_ref[0] + pool_ref[1]
    h = jnp.dot(xp, w1x_ref[...], preferred_element_type=jnp.float32)
    h = h + f_ref[...] * wf_ref[...]
    h = h + t_ref[...] * wt_ref[...]
    h = h + b1_ref[...]
    h = jnp.where(h >= 0, h, 0.01 * h)
    out_ref[...] = (
        jnp.dot(h, w2_ref[...], preferred_element_type=jnp.float32) + b2_ref[...]
    )


def kernel(x, edge_index, batch, feature_index, threshold, W1, b1, W2, b2):
    del edge_index
    ids = batch.astype(jnp.int32)
    zeros = jnp.zeros((ROWS_PER_TILE, D), jnp.float32)
    partials = _segsum_sc(x, ids, zeros)

    w1x = W1[:D]
    wf = W1[D:D + 1]
    wt = W1[D + 1:D + 2]
    w2p = jnp.pad(W2, ((0, 0), (0, D - W2.shape[1])))
    b2p = jnp.pad(b2, (0, D - b2.shape[0]))[None, :]

    out = pl.pallas_call(
        _mlp_tc,
        out_shape=jax.ShapeDtypeStruct((B_SEG, D), jnp.float32),
    )(partials, feature_index[:, None], threshold[:, None], w1x, wf, wt,
      b1[None, :], w2p, b2p)
    return out[:, :2]


# speedup vs baseline: 5.5096x; 1.0458x over previous
import functools

import jax
import jax.numpy as jnp
from jax import lax
from jax.experimental import pallas as pl
from jax.experimental.pallas import tpu as pltpu
from jax.experimental.pallas import tpu_sc as plsc

N_NODES = 100000
D = 128
B_SEG = 2048
HIDDEN = 64

NC, NS = 2, 16
NW = NC * NS
SCAT = 128
UNIT = 384
IDR = UNIT // SCAT
N_UNITS = N_NODES // UNIT
TAIL_A = 99840
TAIL_B = 99968
TAIL_B_LEN = N_NODES - TAIL_B
MAXK = (N_UNITS + NW - 1) // NW
ROWS_PER_TILE = B_SEG // NS

_mesh = plsc.VectorSubcoreMesh(
    core_axis_name="c", subcore_axis_name="s", num_cores=NC, num_subcores=NS
)


@functools.partial(
    pl.kernel,
    out_type=jax.ShapeDtypeStruct((NC, B_SEG, D), jnp.float32),
    mesh=_mesh,
    scratch_types=[
        pltpu.VMEM((UNIT, D), jnp.float32),
        pltpu.VMEM((UNIT, D), jnp.float32),
        pltpu.VMEM((IDR, SCAT), jnp.int32),
        pltpu.VMEM((IDR, SCAT), jnp.int32),
        pltpu.VMEM((TAIL_B_LEN,), jnp.int32),
        pltpu.SemaphoreType.DMA,
        pltpu.SemaphoreType.DMA,
        pltpu.VMEM_SHARED((B_SEG, D), jnp.float32),
    ],
)
def _segsum_sc(x_hbm, ids3_hbm, ids_hbm, zeros_hbm, out_hbm,
               xb0, xb1, id0, id1, idxt, sem0, sem1, acc):
    c = lax.axis_index("c")
    s = lax.axis_index("s")
    w = s * NC + c

    pltpu.sync_copy(zeros_hbm, acc.at[pl.ds(s * ROWS_PER_TILE, ROWS_PER_TILE), :])
    plsc.subcore_barrier()

    def start(u, xb, idb, sem):
        pltpu.async_copy(x_hbm.at[pl.ds(u * UNIT, UNIT), :], xb, sem)
        pltpu.async_copy(ids3_hbm.at[u], idb, sem)

    def finish(u, xb, idb, sem):
        pltpu.make_async_copy(x_hbm.at[pl.ds(u * UNIT, UNIT), :], xb, sem).wait()
        pltpu.make_async_copy(ids3_hbm.at[u], idb, sem).wait()
        for j in range(IDR):
            pltpu.sync_copy(xb.at[pl.ds(j * SCAT, SCAT), :],
                            acc.at[idb.at[j]], add=True)

    start(w, xb0, id0, sem0)

    def body2(k2, carry):
        u_a = w + (2 * k2) * NW
        u_b = u_a + NW
        u_c = u_b + NW

        @pl.when(u_b < N_UNITS)
        def _():
            start(u_b, xb1, id1, sem1)

        @pl.when(u_a < N_UNITS)
        def _():
            finish(u_a, xb0, id0, sem0)

        @pl.when(u_c < N_UNITS)
        def _():
            start(u_c, xb0, id0, sem0)

        @pl.when(u_b < N_UNITS)
        def _():
            finish(u_b, xb1, id1, sem1)

        return carry

    lax.fori_loop(0, (MAXK + 1) // 2, body2, 0)

    @pl.when(w == 0)
    def _():
        pltpu.sync_copy(x_hbm.at[pl.ds(TAIL_A, SCAT), :], xb0.at[pl.ds(0, SCAT), :])
        pltpu.sync_copy(ids_hbm.at[pl.ds(TAIL_A, SCAT)], id0.at[0])
        pltpu.sync_copy(xb0.at[pl.ds(0, SCAT), :], acc.at[id0.at[0]], add=True)

    @pl.when(w == 1)
    def _():
        pltpu.sync_copy(x_hbm.at[pl.ds(TAIL_B, TAIL_B_LEN), :],
                        xb1.at[pl.ds(0, TAIL_B_LEN), :])
        pltpu.sync_copy(ids_hbm.at[pl.ds(TAIL_B, TAIL_B_LEN)], idxt)
        pltpu.sync_copy(xb1.at[pl.ds(0, TAIL_B_LEN), :], acc.at[idxt], add=True)

    plsc.subcore_barrier()
    sl = pl.ds(s * ROWS_PER_TILE, ROWS_PER_TILE)
    pltpu.sync_copy(acc.at[sl, :], out_hbm.at[c, sl, :])


def _mlp_tc(pool_ref, f_ref, t_ref, w1x_ref, wf_ref, wt_ref, b1_ref, w2_ref,
            b2_ref, out_ref):
    xp = pool_ref[0] + pool_ref[1]
    h = jnp.dot(xp, w1x_ref[...], preferred_element_type=jnp.float32)
    h = h + f_ref[...] * wf_ref[...]
    h = h + t_ref[...] * wt_ref[...]
    h = h + b1_ref[...]
    h = jnp.where(h >= 0, h, 0.01 * h)
    out_ref[...] = (
        jnp.dot(h, w2_ref[...], preferred_element_type=jnp.float32) + b2_ref[...]
    )


def kernel(x, edge_index, batch, feature_index, threshold, W1, b1, W2, b2):
    del edge_index
    ids = batch.astype(jnp.int32)
    ids3 = ids[:N_UNITS * UNIT].reshape(N_UNITS, IDR, SCAT)
    zeros = jnp.zeros((ROWS_PER_TILE, D), jnp.float32)
    partials = _segsum_sc(x, ids3, ids, zeros)

    w1x = W1[:D]
    wf = W1[D:D + 1]
    wt = W1[D + 1:D + 2]
    w2p = jnp.pad(W2, ((0, 0), (0, D - W2.shape[1])))
    b2p = jnp.pad(b2, (0, D - b2.shape[0]))[None, :]

    out = pl.pallas_call(
        _mlp_tc,
        out_shape=jax.ShapeDtypeStruct((B_SEG, D), jnp.float32),
    )(partials, feature_index[:, None], threshold[:, None], w1x, wf, wt,
      b1[None, :], w2p, b2p)
    return out[:, :2]


# speedup vs baseline: 5.9235x; 1.0751x over previous
import functools

import jax
import jax.numpy as jnp
from jax import lax
from jax.experimental import pallas as pl
from jax.experimental.pallas import tpu as pltpu
from jax.experimental.pallas import tpu_sc as plsc

N_NODES = 100000
D = 128
B_SEG = 2048
HIDDEN = 64
OUT_DIM = 2

NC, NS = 2, 16
NW = NC * NS
SCAT = 128
UNIT = 256
IDR = UNIT // SCAT
NSLOT = 3
N_UNITS = N_NODES // UNIT
TAIL_A = 99840
TAIL_B = 99968
TAIL_B_LEN = N_NODES - TAIL_B
MAXK = (N_UNITS + NW - 1) // NW
ROWS_PER_TILE = B_SEG // NS

_mesh = plsc.VectorSubcoreMesh(
    core_axis_name="c", subcore_axis_name="s", num_cores=NC, num_subcores=NS
)

_scratch = (
    [pltpu.VMEM((UNIT, D), jnp.float32) for _ in range(NSLOT)] +
    [pltpu.VMEM((IDR, SCAT), jnp.int32) for _ in range(NSLOT)] +
    [pltpu.VMEM((TAIL_B_LEN,), jnp.int32)] +
    [pltpu.SemaphoreType.DMA for _ in range(2 * NSLOT)] +
    [pltpu.VMEM_SHARED((B_SEG, D), jnp.float32)]
)


@functools.partial(
    pl.kernel,
    out_type=jax.ShapeDtypeStruct((NC, B_SEG, D), jnp.float32),
    mesh=_mesh,
    scratch_types=_scratch,
)
def _segsum_sc(x_hbm, ids3_hbm, ids_hbm, zeros_hbm, out_hbm,
               xb0, xb1, xb2, id0, id1, id2, idxt,
               dsem0, dsem1, dsem2, ssem0, ssem1, ssem2, acc):
    xb = (xb0, xb1, xb2)
    idb = (id0, id1, id2)
    dsem = (dsem0, dsem1, dsem2)
    ssem = (ssem0, ssem1, ssem2)

    c = lax.axis_index("c")
    s = lax.axis_index("s")
    w = s * NC + c

    pltpu.sync_copy(zeros_hbm, acc.at[pl.ds(s * ROWS_PER_TILE, ROWS_PER_TILE), :])
    plsc.subcore_barrier()

    def drain_scatter(i):
        for j in range(IDR):
            pltpu.make_async_copy(xb[i].at[pl.ds(j * SCAT, SCAT), :],
                                  acc.at[idb[i].at[j]], ssem[i]).wait()

    def fire_dma(u, i, drain):
        @pl.when(u < N_UNITS)
        def _():
            if drain:
                drain_scatter(i)
            pltpu.async_copy(x_hbm.at[pl.ds(u * UNIT, UNIT), :], xb[i], dsem[i])
            pltpu.async_copy(ids3_hbm.at[u], idb[i], dsem[i])

    def process(u, i):
        @pl.when(u < N_UNITS)
        def _():
            pltpu.make_async_copy(x_hbm.at[pl.ds(u * UNIT, UNIT), :],
                                  xb[i], dsem[i]).wait()
            pltpu.make_async_copy(ids3_hbm.at[u], idb[i], dsem[i]).wait()
            for j in range(IDR):
                pltpu.async_copy(xb[i].at[pl.ds(j * SCAT, SCAT), :],
                                 acc.at[idb[i].at[j]], ssem[i], add=True)

    for k in range(NSLOT):
        fire_dma(w + k * NW, k % NSLOT, drain=False)

    def body(e, carry):
        for r in range(NSLOT):
            u = w + (NSLOT * e + r) * NW
            process(u, r)
            fire_dma(u + NSLOT * NW, r, drain=True)
        return carry

    lax.fori_loop(0, MAXK // NSLOT, body, 0)
    for k in range(NSLOT * (MAXK // NSLOT), MAXK):
        process(w + k * NW, k % NSLOT)

    for k in range(MAXK - NSLOT, MAXK):
        u = w + k * NW
        i = k % NSLOT

        @pl.when(u < N_UNITS)
        def _():
            drain_scatter(i)

    @pl.when(w == 0)
    def _():
        pltpu.sync_copy(x_hbm.at[pl.ds(TAIL_A, SCAT), :], xb0.at[pl.ds(0, SCAT), :])
        pltpu.sync_copy(ids_hbm.at[pl.ds(TAIL_A, SCAT)], id0.at[0])
        pltpu.sync_copy(xb0.at[pl.ds(0, SCAT), :], acc.at[id0.at[0]], add=True)

    @pl.when(w == 1)
    def _():
        pltpu.sync_copy(x_hbm.at[pl.ds(TAIL_B, TAIL_B_LEN), :],
                        xb1.at[pl.ds(0, TAIL_B_LEN), :])
        pltpu.sync_copy(ids_hbm.at[pl.ds(TAIL_B, TAIL_B_LEN)], idxt)
        pltpu.sync_copy(xb1.at[pl.ds(0, TAIL_B_LEN), :], acc.at[idxt], add=True)

    plsc.subcore_barrier()
    sl = pl.ds(s * ROWS_PER_TILE, ROWS_PER_TILE)
    pltpu.sync_copy(acc.at[sl, :], out_hbm.at[c, sl, :])


def _mlp_tc(pool_ref, f_ref, t_ref, w1x_ref, wf_ref, wt_ref, b1_ref, w2_ref,
            b2_ref, out_ref):
    xp = pool_ref[0] + pool_ref[1]
    h = jnp.dot(xp, w1x_ref[...], preferred_element_type=jnp.float32)
    h = h + f_ref[...] * wf_ref[...]
    h = h + t_ref[...] * wt_ref[...]
    h = h + b1_ref[...]
    h = jnp.where(h >= 0, h, 0.01 * h)
    out_ref[...] = (
        jnp.dot(h, w2_ref[...], preferred_element_type=jnp.float32) + b2_ref[...]
    )


def kernel(x, edge_index, batch, feature_index, threshold, W1, b1, W2, b2):
    del edge_index
    ids = batch.astype(jnp.int32)
    ids3 = ids[:N_UNITS * UNIT].reshape(N_UNITS, IDR, SCAT)
    zeros = jnp.zeros((ROWS_PER_TILE, D), jnp.float32)
    partials = _segsum_sc(x, ids3, ids, zeros)

    w1x = W1[:D]
    wf = W1[D:D + 1]
    wt = W1[D + 1:D + 2]

    return pl.pallas_call(
        _mlp_tc,
        out_shape=jax.ShapeDtypeStruct((B_SEG, OUT_DIM), jnp.float32),
    )(partials, feature_index[:, None], threshold[:, None], w1x, wf, wt,
      b1[None, :], W2, b2[None, :])


# speedup vs baseline: 5.9637x; 1.0068x over previous
import functools

import jax
import jax.numpy as jnp
from jax import lax
from jax.experimental import pallas as pl
from jax.experimental.pallas import tpu as pltpu
from jax.experimental.pallas import tpu_sc as plsc

N_NODES = 100000
D = 128
B_SEG = 2048
HIDDEN = 64
OUT_DIM = 2

NC, NS = 2, 16
NW = NC * NS
SCAT = 128
UNIT = 256
IDR = UNIT // SCAT
NSLOT = 3
N_UNITS = N_NODES // UNIT
TAIL_A = 99840
TAIL_B = 99968
TAIL_B_LEN = N_NODES - TAIL_B
MAXK = (N_UNITS + NW - 1) // NW
ROWS_PER_TILE = B_SEG // NS

_mesh = plsc.VectorSubcoreMesh(
    core_axis_name="c", subcore_axis_name="s", num_cores=NC, num_subcores=NS
)

_scratch = (
    [pltpu.VMEM((UNIT, D), jnp.float32) for _ in range(NSLOT)] +
    [pltpu.VMEM((IDR, SCAT), jnp.int32) for _ in range(NSLOT)] +
    [pltpu.VMEM((TAIL_B_LEN,), jnp.int32)] +
    [pltpu.SemaphoreType.DMA for _ in range(2 * NSLOT)] +
    [pltpu.VMEM_SHARED((B_SEG, D), jnp.float32)]
)


@functools.partial(
    pl.kernel,
    out_type=jax.ShapeDtypeStruct((NC, B_SEG, D), jnp.float32),
    mesh=_mesh,
    scratch_types=_scratch,
)
def _segsum_sc(x_hbm, ids3_hbm, ids_hbm, zeros_hbm, out_hbm,
               xb0, xb1, xb2, id0, id1, id2, idxt,
               dsem0, dsem1, dsem2, ssem0, ssem1, ssem2, acc):
    xb = (xb0, xb1, xb2)
    idb = (id0, id1, id2)
    dsem = (dsem0, dsem1, dsem2)
    ssem = (ssem0, ssem1, ssem2)

    c = lax.axis_index("c")
    s = lax.axis_index("s")
    w = s * NC + c

    def drain_scatter(i):
        for j in range(IDR):
            pltpu.make_async_copy(xb[i].at[pl.ds(j * SCAT, SCAT), :],
                                  acc.at[idb[i].at[j]], ssem[i]).wait()

    def fire_dma(u, i, drain):
        @pl.when(u < N_UNITS)
        def _():
            if drain:
                drain_scatter(i)
            pltpu.async_copy(x_hbm.at[pl.ds(u * UNIT, UNIT), :], xb[i], dsem[i])
            pltpu.async_copy(ids3_hbm.at[u], idb[i], dsem[i])

    def process(u, i):
        @pl.when(u < N_UNITS)
        def _():
            pltpu.make_async_copy(x_hbm.at[pl.ds(u * UNIT, UNIT), :],
                                  xb[i], dsem[i]).wait()
            pltpu.make_async_copy(ids3_hbm.at[u], idb[i], dsem[i]).wait()
            for j in range(IDR):
                pltpu.async_copy(xb[i].at[pl.ds(j * SCAT, SCAT), :],
                                 acc.at[idb[i].at[j]], ssem[i], add=True)

    for k in range(NSLOT):
        fire_dma(w + k * NW, k % NSLOT, drain=False)

    pltpu.sync_copy(zeros_hbm, acc.at[pl.ds(s * ROWS_PER_TILE, ROWS_PER_TILE), :])
    plsc.subcore_barrier()

    def body(e, carry):
        for r in range(NSLOT):
            u = w + (NSLOT * e + r) * NW
            process(u, r)
            fire_dma(u + NSLOT * NW, r, drain=True)
        return carry

    lax.fori_loop(0, MAXK // NSLOT, body, 0)
    for k in range(NSLOT * (MAXK // NSLOT), MAXK):
        process(w + k * NW, k % NSLOT)

    for k in range(MAXK - NSLOT, MAXK):
        u = w + k * NW
        i = k % NSLOT

        @pl.when(u < N_UNITS)
        def _():
            drain_scatter(i)

    @pl.when(w == 1)
    def _():
        pltpu.sync_copy(x_hbm.at[pl.ds(TAIL_A, SCAT), :], xb0.at[pl.ds(0, SCAT), :])
        pltpu.sync_copy(ids_hbm.at[pl.ds(TAIL_A, SCAT)], id0.at[0])
        pltpu.sync_copy(xb0.at[pl.ds(0, SCAT), :], acc.at[id0.at[0]], add=True)

    @pl.when(w == 3)
    def _():
        pltpu.sync_copy(x_hbm.at[pl.ds(TAIL_B, TAIL_B_LEN), :],
                        xb1.at[pl.ds(0, TAIL_B_LEN), :])
        pltpu.sync_copy(ids_hbm.at[pl.ds(TAIL_B, TAIL_B_LEN)], idxt)
        pltpu.sync_copy(xb1.at[pl.ds(0, TAIL_B_LEN), :], acc.at[idxt], add=True)

    plsc.subcore_barrier()
    sl = pl.ds(s * ROWS_PER_TILE, ROWS_PER_TILE)
    pltpu.sync_copy(acc.at[sl, :], out_hbm.at[c, sl, :])


def _mlp_tc(pool_ref, f_ref, t_ref, w1x_ref, wf_ref, wt_ref, b1_ref, w2_ref,
            b2_ref, out_ref):
    xp = pool_ref[0] + pool_ref[1]
    h = jnp.dot(xp, w1x_ref[...], preferred_element_type=jnp.float32)
    h = h + f_ref[...] * wf_ref[...]
    h = h + t_ref[...] * wt_ref[...]
    h = h + b1_ref[...]
    h = jnp.where(h >= 0, h, 0.01 * h)
    out_ref[...] = (
        jnp.dot(h, w2_ref[...], preferred_element_type=jnp.float32) + b2_ref[...]
    )


def kernel(x, edge_index, batch, feature_index, threshold, W1, b1, W2, b2):
    del edge_index
    ids = batch.astype(jnp.int32)
    ids3 = ids[:N_UNITS * UNIT].reshape(N_UNITS, IDR, SCAT)
    zeros = jnp.zeros((ROWS_PER_TILE, D), jnp.float32)
    partials = _segsum_sc(x, ids3, ids, zeros)

    w1x = W1[:D]
    wf = W1[D:D + 1]
    wt = W1[D + 1:D + 2]

    return pl.pallas_call(
        _mlp_tc,
        out_shape=jax.ShapeDtypeStruct((B_SEG, OUT_DIM), jnp.float32),
    )(partials, feature_index[:, None], threshold[:, None], w1x, wf, wt,
      b1[None, :], W2, b2[None, :])


# speedup vs baseline: 6.1947x; 1.0387x over previous
import functools

import jax
import jax.numpy as jnp
from jax import lax
from jax.experimental import pallas as pl
from jax.experimental.pallas import tpu as pltpu
from jax.experimental.pallas import tpu_sc as plsc

N_NODES = 100000
D = 128
B_SEG = 2048
HIDDEN = 64
OUT_DIM = 2

NC, NS = 2, 16
NW = NC * NS
UNIT = 128
NSLOT = 6
N_UNITS = N_NODES // UNIT
TAIL_B = N_UNITS * UNIT
TAIL_B_LEN = N_NODES - TAIL_B
MAXK = (N_UNITS + NW - 1) // NW
ROWS_PER_TILE = B_SEG // NS
MLP_BLK = 512

_mesh = plsc.VectorSubcoreMesh(
    core_axis_name="c", subcore_axis_name="s", num_cores=NC, num_subcores=NS
)

_scratch = (
    [pltpu.VMEM((UNIT, D), jnp.float32) for _ in range(NSLOT)] +
    [pltpu.VMEM((1, UNIT), jnp.int32) for _ in range(NSLOT)] +
    [pltpu.VMEM((TAIL_B_LEN,), jnp.int32)] +
    [pltpu.SemaphoreType.DMA for _ in range(2 * NSLOT)] +
    [pltpu.VMEM_SHARED((B_SEG, D), jnp.float32)]
)


@functools.partial(
    pl.kernel,
    out_type=jax.ShapeDtypeStruct((NC, B_SEG, D), jnp.float32),
    mesh=_mesh,
    scratch_types=_scratch,
)
def _segsum_sc(x_hbm, ids_hbm, zeros_hbm, out_hbm,
               xb0, xb1, xb2, xb3, xb4, xb5,
               id0, id1, id2, id3, id4, id5, idxt,
               ds0, ds1, ds2, ds3, ds4, ds5,
               ss0, ss1, ss2, ss3, ss4, ss5, acc):
    xb = (xb0, xb1, xb2, xb3, xb4, xb5)
    idb = (id0, id1, id2, id3, id4, id5)
    dsem = (ds0, ds1, ds2, ds3, ds4, ds5)
    ssem = (ss0, ss1, ss2, ss3, ss4, ss5)

    c = lax.axis_index("c")
    s = lax.axis_index("s")
    w = s * NC + c

    def drain_scatter(i):
        pltpu.make_async_copy(xb[i], acc.at[idb[i].at[0]], ssem[i]).wait()

    def fire_dma(u, i, drain):
        @pl.when(u < N_UNITS)
        def _():
            if drain:
                drain_scatter(i)
            pltpu.async_copy(x_hbm.at[pl.ds(u * UNIT, UNIT), :], xb[i], dsem[i])
            pltpu.async_copy(ids_hbm.at[pl.ds(u * UNIT, UNIT)], idb[i].at[0],
                             dsem[i])

    def process(u, i):
        @pl.when(u < N_UNITS)
        def _():
            pltpu.make_async_copy(x_hbm.at[pl.ds(u * UNIT, UNIT), :],
                                  xb[i], dsem[i]).wait()
            pltpu.make_async_copy(ids_hbm.at[pl.ds(u * UNIT, UNIT)],
                                  idb[i].at[0], dsem[i]).wait()
            pltpu.async_copy(xb[i], acc.at[idb[i].at[0]], ssem[i], add=True)

    for k in range(NSLOT):
        fire_dma(w + k * NW, k % NSLOT, drain=False)

    pltpu.sync_copy(zeros_hbm, acc.at[pl.ds(s * ROWS_PER_TILE, ROWS_PER_TILE), :])
    plsc.subcore_barrier()

    def body(e, carry):
        for r in range(NSLOT):
            u = w + (NSLOT * e + r) * NW
            process(u, r)
            fire_dma(u + NSLOT * NW, r, drain=True)
        return carry

    lax.fori_loop(0, MAXK // NSLOT, body, 0)
    for k in range(NSLOT * (MAXK // NSLOT), MAXK):
        process(w + k * NW, k % NSLOT)

    for k in range(MAXK - NSLOT, MAXK):
        u = w + k * NW
        i = k % NSLOT

        @pl.when(u < N_UNITS)
        def _():
            drain_scatter(i)

    @pl.when(w == 1)
    def _():
        pltpu.sync_copy(x_hbm.at[pl.ds(TAIL_B, TAIL_B_LEN), :],
                        xb0.at[pl.ds(0, TAIL_B_LEN), :])
        pltpu.sync_copy(ids_hbm.at[pl.ds(TAIL_B, TAIL_B_LEN)], idxt)
        pltpu.sync_copy(xb0.at[pl.ds(0, TAIL_B_LEN), :], acc.at[idxt], add=True)

    plsc.subcore_barrier()
    sl = pl.ds(s * ROWS_PER_TILE, ROWS_PER_TILE)
    pltpu.sync_copy(acc.at[sl, :], out_hbm.at[c, sl, :])


def _mlp_tc(pool_ref, f_ref, t_ref, w1x_ref, wf_ref, wt_ref, b1_ref, w2_ref,
            b2_ref, out_ref):
    xp = pool_ref[0] + pool_ref[1]
    h = jnp.dot(xp, w1x_ref[...], preferred_element_type=jnp.float32)
    h = h + f_ref[...] * wf_ref[...]
    h = h + t_ref[...] * wt_ref[...]
    h = h + b1_ref[...]
    h = jnp.where(h >= 0, h, 0.01 * h)
    out_ref[...] = (
        jnp.dot(h, w2_ref[...], preferred_element_type=jnp.float32) + b2_ref[...]
    )


def kernel(x, edge_index, batch, feature_index, threshold, W1, b1, W2, b2):
    del edge_index
    ids = batch.astype(jnp.int32)
    zeros = jnp.zeros((ROWS_PER_TILE, D), jnp.float32)
    partials = _segsum_sc(x, ids, zeros)

    w1x = W1[:D]
    wf = W1[D:D + 1]
    wt = W1[D + 1:D + 2]

    return pl.pallas_call(
        _mlp_tc,
        grid=(B_SEG // MLP_BLK,),
        in_specs=[
            pl.BlockSpec((NC, MLP_BLK, D), lambda i: (0, i, 0)),
            pl.BlockSpec((MLP_BLK, 1), lambda i: (i, 0)),
            pl.BlockSpec((MLP_BLK, 1), lambda i: (i, 0)),
            pl.BlockSpec((D, HIDDEN), lambda i: (0, 0)),
            pl.BlockSpec((1, HIDDEN), lambda i: (0, 0)),
            pl.BlockSpec((1, HIDDEN), lambda i: (0, 0)),
            pl.BlockSpec((1, HIDDEN), lambda i: (0, 0)),
            pl.BlockSpec((HIDDEN, OUT_DIM), lambda i: (0, 0)),
            pl.BlockSpec((1, OUT_DIM), lambda i: (0, 0)),
        ],
        out_specs=pl.BlockSpec((MLP_BLK, OUT_DIM), lambda i: (i, 0)),
        out_shape=jax.ShapeDtypeStruct((B_SEG, OUT_DIM), jnp.float32),
    )(partials, feature_index[:, None], threshold[:, None], w1x, wf, wt,
      b1[None, :], W2, b2[None, :])


# speedup vs baseline: 6.2752x; 1.0130x over previous
import functools

import jax
import jax.numpy as jnp
from jax import lax
from jax.experimental import pallas as pl
from jax.experimental.pallas import tpu as pltpu
from jax.experimental.pallas import tpu_sc as plsc

N_NODES = 100000
D = 128
B_SEG = 2048
HIDDEN = 64
OUT_DIM = 2

NC, NS = 2, 16
NW = NC * NS
UNIT = 128
NSLOT = 6
N_UNITS = N_NODES // UNIT
TAIL_B = N_UNITS * UNIT
TAIL_B_LEN = N_NODES - TAIL_B
MAXK = (N_UNITS + NW - 1) // NW
ROWS_PER_TILE = B_SEG // NS
MLP_BLK = 512

_mesh = plsc.VectorSubcoreMesh(
    core_axis_name="c", subcore_axis_name="s", num_cores=NC, num_subcores=NS
)

_scratch = (
    [pltpu.VMEM((UNIT, D), jnp.float32) for _ in range(NSLOT)] +
    [pltpu.VMEM((1, UNIT), jnp.int32) for _ in range(NSLOT)] +
    [pltpu.VMEM((TAIL_B_LEN,), jnp.int32)] +
    [pltpu.SemaphoreType.DMA for _ in range(2 * NSLOT)] +
    [pltpu.VMEM_SHARED((B_SEG, D), jnp.float32)]
)


@functools.partial(
    pl.kernel,
    out_type=jax.ShapeDtypeStruct((NC, B_SEG, D), jnp.float32),
    mesh=_mesh,
    scratch_types=_scratch,
)
def _segsum_sc(x_hbm, ids_hbm, zeros_hbm, out_hbm,
               xb0, xb1, xb2, xb3, xb4, xb5,
               id0, id1, id2, id3, id4, id5, idxt,
               ds0, ds1, ds2, ds3, ds4, ds5,
               ss0, ss1, ss2, ss3, ss4, ss5, acc):
    xb = (xb0, xb1, xb2, xb3, xb4, xb5)
    idb = (id0, id1, id2, id3, id4, id5)
    dsem = (ds0, ds1, ds2, ds3, ds4, ds5)
    ssem = (ss0, ss1, ss2, ss3, ss4, ss5)

    c = lax.axis_index("c")
    s = lax.axis_index("s")
    w = s * NC + c

    def drain_scatter(i):
        pltpu.make_async_copy(xb[i], acc.at[idb[i].at[0]], ssem[i]).wait()

    def fire_dma(u, i, drain):
        @pl.when(u < N_UNITS)
        def _():
            if drain:
                drain_scatter(i)
            pltpu.async_copy(x_hbm.at[pl.ds(u * UNIT, UNIT), :], xb[i], dsem[i])
            pltpu.async_copy(ids_hbm.at[pl.ds(u * UNIT, UNIT)], idb[i].at[0],
                             dsem[i])

    def process(u, i):
        @pl.when(u < N_UNITS)
        def _():
            pltpu.make_async_copy(x_hbm.at[pl.ds(u * UNIT, UNIT), :],
                                  xb[i], dsem[i]).wait()
            pltpu.make_async_copy(ids_hbm.at[pl.ds(u * UNIT, UNIT)],
                                  idb[i].at[0], dsem[i]).wait()
            pltpu.async_copy(xb[i], acc.at[idb[i].at[0]], ssem[i], add=True)

    for k in range(NSLOT):
        fire_dma(w + k * NW, k % NSLOT, drain=False)

    pltpu.sync_copy(zeros_hbm, acc.at[pl.ds(s * ROWS_PER_TILE, ROWS_PER_TILE), :])
    plsc.subcore_barrier()

    def body(e, carry):
        for r in range(NSLOT):
            u = w + (NSLOT * e + r) * NW
            process(u, r)
            fire_dma(u + NSLOT * NW, r, drain=True)
        return carry

    lax.fori_loop(0, MAXK // NSLOT, body, 0)
    for k in range(NSLOT * (MAXK // NSLOT), MAXK):
        process(w + k * NW, k % NSLOT)

    for k in range(MAXK - NSLOT, MAXK):
        u = w + k * NW
        i = k % NSLOT

        @pl.when(u < N_UNITS)
        def _():
            drain_scatter(i)

    @pl.when(w == 1)
    def _():
        pltpu.sync_copy(x_hbm.at[pl.ds(TAIL_B, TAIL_B_LEN), :],
                        xb0.at[pl.ds(0, TAIL_B_LEN), :])
        pltpu.sync_copy(ids_hbm.at[pl.ds(TAIL_B, TAIL_B_LEN)], idxt)
        pltpu.sync_copy(xb0.at[pl.ds(0, TAIL_B_LEN), :], acc.at[idxt], add=True)

    plsc.subcore_barrier()
    sl = pl.ds(s * ROWS_PER_TILE, ROWS_PER_TILE)
    pltpu.sync_copy(acc.at[sl, :], out_hbm.at[c, sl, :])


def _mlp_tc(pool_ref, f_ref, t_ref, w1x_ref, wf_ref, wt_ref, b1_ref, w2_ref,
            b2_ref, out_ref):
    xp = pool_ref[0] + pool_ref[1]
    h = jnp.dot(xp, w1x_ref[...], preferred_element_type=jnp.float32)
    h = h + f_ref[...] * wf_ref[...]
    h = h + t_ref[...] * wt_ref[...]
    h = h + b1_ref[...]
    h = jnp.where(h >= 0, h, 0.01 * h)
    out_ref[...] = (
        jnp.dot(h, w2_ref[...], preferred_element_type=jnp.float32) + b2_ref[...]
    )


def kernel(x, edge_index, batch, feature_index, threshold, W1, b1, W2, b2):
    del edge_index
    ids = batch.astype(jnp.int32)
    zeros = jnp.zeros((ROWS_PER_TILE, D), jnp.float32)
    partials = _segsum_sc(x, ids, zeros)

    w1x = W1[:D]
    wf = W1[D:D + 1]
    wt = W1[D + 1:D + 2]

    return pl.pallas_call(
        _mlp_tc,
        out_shape=jax.ShapeDtypeStruct((B_SEG, OUT_DIM), jnp.float32),
    )(partials, feature_index[:, None], threshold[:, None], w1x, wf, wt,
      b1[None, :], W2, b2[None, :])


# speedup vs baseline: 6.3229x; 1.0076x over previous
import functools

import jax
import jax.numpy as jnp
from jax import lax
from jax.experimental import pallas as pl
from jax.experimental.pallas import tpu as pltpu
from jax.experimental.pallas import tpu_sc as plsc

N_NODES = 100000
D = 128
B_SEG = 2048
HIDDEN = 64
OUT_DIM = 2

NC, NS = 2, 16
NW = NC * NS
UNIT = 128
NSLOT = 6
N_UNITS = N_NODES // UNIT
TAIL_B = N_UNITS * UNIT
TAIL_B_LEN = N_NODES - TAIL_B
MAXK = (N_UNITS + NW - 1) // NW
ROWS_PER_TILE = B_SEG // NS
MLP_BLK = 512

_mesh = plsc.VectorSubcoreMesh(
    core_axis_name="c", subcore_axis_name="s", num_cores=NC, num_subcores=NS
)

_scratch = (
    [pltpu.VMEM((UNIT, D), jnp.float32) for _ in range(NSLOT)] +
    [pltpu.VMEM((1, UNIT), jnp.int32) for _ in range(NSLOT)] +
    [pltpu.VMEM((TAIL_B_LEN,), jnp.int32)] +
    [pltpu.SemaphoreType.DMA for _ in range(2 * NSLOT)] +
    [pltpu.VMEM_SHARED((B_SEG, D), jnp.float32)]
)


@functools.partial(
    pl.kernel,
    out_type=jax.ShapeDtypeStruct((NC, B_SEG, D), jnp.float32),
    mesh=_mesh,
    scratch_types=_scratch,
)
def _segsum_sc(x_hbm, ids_hbm, zeros_hbm, out_hbm,
               xb0, xb1, xb2, xb3, xb4, xb5,
               id0, id1, id2, id3, id4, id5, idxt,
               ds0, ds1, ds2, ds3, ds4, ds5,
               ss0, ss1, ss2, ss3, ss4, ss5, acc):
    xb = (xb0, xb1, xb2, xb3, xb4, xb5)
    idb = (id0, id1, id2, id3, id4, id5)
    dsem = (ds0, ds1, ds2, ds3, ds4, ds5)
    ssem = (ss0, ss1, ss2, ss3, ss4, ss5)

    c = lax.axis_index("c")
    s = lax.axis_index("s")
    w = s * NC + c

    def drain_scatter(i):
        pltpu.make_async_copy(xb[i], acc.at[idb[i].at[0]], ssem[i]).wait()

    def fire_dma(u, i, drain):
        @pl.when(u < N_UNITS)
        def _():
            if drain:
                drain_scatter(i)
            pltpu.async_copy(x_hbm.at[pl.ds(u * UNIT, UNIT), :], xb[i], dsem[i])
            pltpu.async_copy(ids_hbm.at[pl.ds(u * UNIT, UNIT)], idb[i].at[0],
                             dsem[i])

    def process(u, i):
        @pl.when(u < N_UNITS)
        def _():
            pltpu.make_async_copy(x_hbm.at[pl.ds(u * UNIT, UNIT), :],
                                  xb[i], dsem[i]).wait()
            pltpu.make_async_copy(ids_hbm.at[pl.ds(u * UNIT, UNIT)],
                                  idb[i].at[0], dsem[i]).wait()
            pltpu.async_copy(xb[i], acc.at[idb[i].at[0]], ssem[i], add=True)

    PRE = 4

    for k in range(PRE):
        fire_dma(w + k * NW, k % NSLOT, drain=False)

    pltpu.sync_copy(zeros_hbm, acc.at[pl.ds(s * ROWS_PER_TILE, ROWS_PER_TILE), :])
    plsc.subcore_barrier()

    for k in range(NSLOT - PRE):
        process(w + k * NW, k % NSLOT)
        fire_dma(w + (k + PRE) * NW, (k + PRE) % NSLOT, drain=False)

    K0 = NSLOT - PRE
    NBODY = (MAXK - K0) // NSLOT

    def body(e, carry):
        for r in range(NSLOT):
            u = w + (K0 + NSLOT * e + r) * NW
            process(u, (K0 + r) % NSLOT)
            fire_dma(u + PRE * NW, (K0 + r + PRE) % NSLOT, drain=True)
        return carry

    lax.fori_loop(0, NBODY, body, 0)
    for k in range(K0 + NSLOT * NBODY, MAXK):
        process(w + k * NW, k % NSLOT)
        fire_dma(w + (k + PRE) * NW, (k + PRE) % NSLOT, drain=True)

    for k in range(max(0, MAXK - 2 * NSLOT), MAXK):
        u = w + k * NW
        i = k % NSLOT

        @pl.when((u < N_UNITS) & (u + NSLOT * NW >= N_UNITS))
        def _():
            drain_scatter(i)

    @pl.when(w == 1)
    def _():
        pltpu.sync_copy(x_hbm.at[pl.ds(TAIL_B, TAIL_B_LEN), :],
                        xb0.at[pl.ds(0, TAIL_B_LEN), :])
        pltpu.sync_copy(ids_hbm.at[pl.ds(TAIL_B, TAIL_B_LEN)], idxt)
        pltpu.sync_copy(xb0.at[pl.ds(0, TAIL_B_LEN), :], acc.at[idxt], add=True)

    plsc.subcore_barrier()
    sl = pl.ds(s * ROWS_PER_TILE, ROWS_PER_TILE)
    pltpu.sync_copy(acc.at[sl, :], out_hbm.at[c, sl, :])


def _mlp_tc(pool_ref, f_ref, t_ref, w1x_ref, wf_ref, wt_ref, b1_ref, w2_ref,
            b2_ref, out_ref):
    xp = pool_ref[0] + pool_ref[1]
    h = jnp.dot(xp, w1x_ref[...], preferred_element_type=jnp.float32)
    h = h + f_ref[...] * wf_ref[...]
    h = h + t_ref[...] * wt_ref[...]
    h = h + b1_ref[...]
    h = jnp.where(h >= 0, h, 0.01 * h)
    out_ref[...] = (
        jnp.dot(h, w2_ref[...], preferred_element_type=jnp.float32) + b2_ref[...]
    )


def kernel(x, edge_index, batch, feature_index, threshold, W1, b1, W2, b2):
    del edge_index
    ids = batch.astype(jnp.int32)
    zeros = jnp.zeros((ROWS_PER_TILE, D), jnp.float32)
    partials = _segsum_sc(x, ids, zeros)

    w1x = W1[:D]
    wf = W1[D:D + 1]
    wt = W1[D + 1:D + 2]

    return pl.pallas_call(
        _mlp_tc,
        out_shape=jax.ShapeDtypeStruct((B_SEG, OUT_DIM), jnp.float32),
    )(partials, feature_index[:, None], threshold[:, None], w1x, wf, wt,
      b1[None, :], W2, b2[None, :])
